# Initial kernel scaffold; baseline (speedup 1.0000x reference)
import jax
import jax.numpy as jnp
from jax.experimental import pallas as pl


def kernel(input_ids, gene_table, expr_table, W_proj, b_proj, pos_table, ln_gamma, ln_beta):
    raise NotImplementedError("write your pallas kernel here")


# speedup vs baseline: 4.1323x; 4.1323x over previous
import functools

import jax
import jax.numpy as jnp
from jax.experimental import pallas as pl

N_GENES = 100000
GENE_DIM = 64
EXPR_PAD = 64
HIDDEN = 128
BATCH = 64
SEQ = 2048
TOK_BLOCK = 1024


def _precompute_kernel(gene_ref, expr_ref, w_ref, b_ref, pos_ref, g_out_ref, e_out_ref):
    w1 = w_ref[0:GENE_DIM, :]
    w2 = w_ref[GENE_DIM : 2 * GENE_DIM, :]
    g = jax.lax.dot_general(
        gene_ref[...], w1, (((1,), (0,)), ((), ())),
        preferred_element_type=jnp.float32,
    )
    g_out_ref[...] = g + b_ref[...] + pos_ref[...]
    e_out_ref[...] = jax.lax.dot_general(
        expr_ref[...], w2, (((1,), (0,)), ((), ())),
        preferred_element_type=jnp.float32,
    )


def _main_kernel(ids_ref, g_ref, e_ref, gamma_ref, beta_ref, out_ref):
    ids = ids_ref[0, :, :]
    ids_b = jnp.broadcast_to(ids, (EXPR_PAD, TOK_BLOCK))
    vocab_iota = jax.lax.broadcasted_iota(jnp.int32, (EXPR_PAD, TOK_BLOCK), 0)
    onehot = (ids_b == vocab_iota).astype(jnp.float32)
    gathered = jax.lax.dot_general(
        onehot, e_ref[...], (((0,), (0,)), ((), ())),
        preferred_element_type=jnp.float32,
    )
    x = gathered + g_ref[...]
    mean = jnp.mean(x, axis=-1, keepdims=True)
    centered = x - mean
    var = jnp.mean(centered * centered, axis=-1, keepdims=True)
    y = centered * jax.lax.rsqrt(var + 1e-5)
    out_ref[...] = y * gamma_ref[...] + beta_ref[...]


@jax.jit
def kernel(input_ids, gene_table, expr_table, W_proj, b_proj, pos_table, ln_gamma, ln_beta):
    B, S = input_ids.shape
    V, D = expr_table.shape
    H = W_proj.shape[1]

    expr_pad = jnp.zeros((EXPR_PAD, D), dtype=expr_table.dtype).at[:V].set(expr_table)

    g_tab, e_tab = pl.pallas_call(
        _precompute_kernel,
        grid=(1,),
        in_specs=[
            pl.BlockSpec((S, D), lambda i: (0, 0)),
            pl.BlockSpec((EXPR_PAD, D), lambda i: (0, 0)),
            pl.BlockSpec((2 * D, H), lambda i: (0, 0)),
            pl.BlockSpec((1, H), lambda i: (0, 0)),
            pl.BlockSpec((S, H), lambda i: (0, 0)),
        ],
        out_specs=[
            pl.BlockSpec((S, H), lambda i: (0, 0)),
            pl.BlockSpec((EXPR_PAD, H), lambda i: (0, 0)),
        ],
        out_shape=[
            jax.ShapeDtypeStruct((S, H), jnp.float32),
            jax.ShapeDtypeStruct((EXPR_PAD, H), jnp.float32),
        ],
    )(gene_table, expr_pad, W_proj, b_proj.reshape(1, H), pos_table)

    n_blocks = (B * S) // TOK_BLOCK
    blocks_per_seq = S // TOK_BLOCK
    ids3 = input_ids.astype(jnp.int32).reshape(n_blocks, 1, TOK_BLOCK)

    out_flat = pl.pallas_call(
        _main_kernel,
        grid=(n_blocks,),
        in_specs=[
            pl.BlockSpec((1, 1, TOK_BLOCK), lambda i: (i, 0, 0)),
            pl.BlockSpec((TOK_BLOCK, H), lambda i, bps=blocks_per_seq: (i % bps, 0)),
            pl.BlockSpec((EXPR_PAD, H), lambda i: (0, 0)),
            pl.BlockSpec((1, H), lambda i: (0, 0)),
            pl.BlockSpec((1, H), lambda i: (0, 0)),
        ],
        out_specs=pl.BlockSpec((TOK_BLOCK, H), lambda i: (i, 0)),
        out_shape=jax.ShapeDtypeStruct((B * S, H), jnp.float32),
    )(ids3, g_tab, e_tab, ln_gamma.reshape(1, H), ln_beta.reshape(1, H))

    return out_flat.reshape(B, S, H)


# speedup vs baseline: 4.1388x; 1.0016x over previous
import functools

import jax
import jax.numpy as jnp
from jax.experimental import pallas as pl
from jax.experimental.pallas import tpu as pltpu

N_GENES = 100000
GENE_DIM = 64
EXPR_PAD = 64
HIDDEN = 128
BATCH = 64
SEQ = 2048
TOK_BLOCK = 1024


def _precompute_kernel(gene_ref, expr_ref, w_ref, b_ref, pos_ref, g_out_ref, e_out_ref):
    w1 = w_ref[0:GENE_DIM, :]
    w2 = w_ref[GENE_DIM : 2 * GENE_DIM, :]
    g = jax.lax.dot_general(
        gene_ref[...], w1, (((1,), (0,)), ((), ())),
        preferred_element_type=jnp.float32,
    )
    g_out_ref[...] = g + b_ref[...] + pos_ref[...]
    e_out_ref[...] = jax.lax.dot_general(
        expr_ref[...], w2, (((1,), (0,)), ((), ())),
        preferred_element_type=jnp.float32,
    )


def _main_kernel(ids_ref, g_ref, e_ref, gamma_ref, beta_ref, out_ref):
    ids = ids_ref[0, :, :]
    ids_b = jnp.broadcast_to(ids, (EXPR_PAD, TOK_BLOCK))
    vocab_iota = jax.lax.broadcasted_iota(jnp.int32, (EXPR_PAD, TOK_BLOCK), 0)
    onehot = (ids_b == vocab_iota).astype(jnp.float32)
    gathered = jax.lax.dot_general(
        onehot, e_ref[...], (((0,), (0,)), ((), ())),
        preferred_element_type=jnp.float32,
    )
    x = gathered + g_ref[...]
    mean = jnp.mean(x, axis=-1, keepdims=True)
    centered = x - mean
    var = jnp.mean(centered * centered, axis=-1, keepdims=True)
    y = centered * jax.lax.rsqrt(var + 1e-5)
    out_ref[...] = y * gamma_ref[...] + beta_ref[...]


@jax.jit
def kernel(input_ids, gene_table, expr_table, W_proj, b_proj, pos_table, ln_gamma, ln_beta):
    B, S = input_ids.shape
    V, D = expr_table.shape
    H = W_proj.shape[1]

    expr_pad = jnp.zeros((EXPR_PAD, D), dtype=expr_table.dtype).at[:V].set(expr_table)

    g_tab, e_tab = pl.pallas_call(
        _precompute_kernel,
        grid=(1,),
        in_specs=[
            pl.BlockSpec((S, D), lambda i: (0, 0)),
            pl.BlockSpec((EXPR_PAD, D), lambda i: (0, 0)),
            pl.BlockSpec((2 * D, H), lambda i: (0, 0)),
            pl.BlockSpec((1, H), lambda i: (0, 0)),
            pl.BlockSpec((S, H), lambda i: (0, 0)),
        ],
        out_specs=[
            pl.BlockSpec((S, H), lambda i: (0, 0)),
            pl.BlockSpec((EXPR_PAD, H), lambda i: (0, 0)),
        ],
        out_shape=[
            jax.ShapeDtypeStruct((S, H), jnp.float32),
            jax.ShapeDtypeStruct((EXPR_PAD, H), jnp.float32),
        ],
    )(gene_table, expr_pad, W_proj, b_proj.reshape(1, H), pos_table)

    n_blocks = (B * S) // TOK_BLOCK
    blocks_per_seq = S // TOK_BLOCK
    ids3 = input_ids.astype(jnp.int32).reshape(n_blocks, 1, TOK_BLOCK)

    out_flat = pl.pallas_call(
        _main_kernel,
        grid=(n_blocks,),
        in_specs=[
            pl.BlockSpec((1, 1, TOK_BLOCK), lambda i: (i, 0, 0)),
            pl.BlockSpec((TOK_BLOCK, H), lambda i, bps=blocks_per_seq: (i % bps, 0)),
            pl.BlockSpec((EXPR_PAD, H), lambda i: (0, 0)),
            pl.BlockSpec((1, H), lambda i: (0, 0)),
            pl.BlockSpec((1, H), lambda i: (0, 0)),
        ],
        out_specs=pl.BlockSpec((TOK_BLOCK, H), lambda i: (i, 0)),
        out_shape=jax.ShapeDtypeStruct((B * S, H), jnp.float32),
        compiler_params=pltpu.CompilerParams(
            dimension_semantics=("parallel",),
        ),
    )(ids3, g_tab, e_tab, ln_gamma.reshape(1, H), ln_beta.reshape(1, H))

    return out_flat.reshape(B, S, H)


# speedup vs baseline: 6.5823x; 1.5904x over previous
import functools

import jax
import jax.numpy as jnp
from jax.experimental import pallas as pl
from jax.experimental.pallas import tpu as pltpu

N_GENES = 100000
GENE_DIM = 64
EXPR_PAD = 64
HIDDEN = 128
BATCH = 64
SEQ = 2048
TOK_BLOCK = 2048


def _precompute_kernel(gene_ref, expr_ref, w_ref, b_ref, pos_ref, g_out_ref, e_out_ref):
    w1 = w_ref[0:GENE_DIM, :]
    w2 = w_ref[GENE_DIM : 2 * GENE_DIM, :]
    g = jax.lax.dot_general(
        gene_ref[...], w1, (((1,), (0,)), ((), ())),
        preferred_element_type=jnp.float32,
    )
    g_out_ref[...] = g + b_ref[...] + pos_ref[...]
    e_out_ref[...] = jax.lax.dot_general(
        expr_ref[...], w2, (((1,), (0,)), ((), ())),
        preferred_element_type=jnp.float32,
    )


def _main_kernel(ids_ref, g_ref, e_ref, gamma_ref, beta_ref, out_ref):
    ids = ids_ref[0, :, :]
    ids_b = jnp.broadcast_to(ids, (EXPR_PAD, TOK_BLOCK))
    vocab_iota = jax.lax.broadcasted_iota(jnp.int32, (EXPR_PAD, TOK_BLOCK), 0)
    onehot = (ids_b == vocab_iota).astype(jnp.float32)
    gathered = jax.lax.dot_general(
        onehot, e_ref[...], (((0,), (0,)), ((), ())),
        preferred_element_type=jnp.float32,
    )
    x = gathered + g_ref[...]
    mean = jnp.mean(x, axis=-1, keepdims=True)
    centered = x - mean
    var = jnp.mean(centered * centered, axis=-1, keepdims=True)
    y = centered * jax.lax.rsqrt(var + 1e-5)
    out_ref[...] = y * gamma_ref[...] + beta_ref[...]


@jax.jit
def kernel(input_ids, gene_table, expr_table, W_proj, b_proj, pos_table, ln_gamma, ln_beta):
    B, S = input_ids.shape
    V, D = expr_table.shape
    H = W_proj.shape[1]

    expr_pad = jnp.zeros((EXPR_PAD, D), dtype=expr_table.dtype).at[:V].set(expr_table)

    g_tab, e_tab = pl.pallas_call(
        _precompute_kernel,
        grid=(1,),
        in_specs=[
            pl.BlockSpec((S, D), lambda i: (0, 0)),
            pl.BlockSpec((EXPR_PAD, D), lambda i: (0, 0)),
            pl.BlockSpec((2 * D, H), lambda i: (0, 0)),
            pl.BlockSpec((1, H), lambda i: (0, 0)),
            pl.BlockSpec((S, H), lambda i: (0, 0)),
        ],
        out_specs=[
            pl.BlockSpec((S, H), lambda i: (0, 0)),
            pl.BlockSpec((EXPR_PAD, H), lambda i: (0, 0)),
        ],
        out_shape=[
            jax.ShapeDtypeStruct((S, H), jnp.float32),
            jax.ShapeDtypeStruct((EXPR_PAD, H), jnp.float32),
        ],
    )(gene_table, expr_pad, W_proj, b_proj.reshape(1, H), pos_table)

    n_blocks = (B * S) // TOK_BLOCK
    blocks_per_seq = S // TOK_BLOCK
    ids3 = input_ids.astype(jnp.int32).reshape(n_blocks, 1, TOK_BLOCK)

    out_flat = pl.pallas_call(
        _main_kernel,
        grid=(n_blocks,),
        in_specs=[
            pl.BlockSpec((1, 1, TOK_BLOCK), lambda i: (i, 0, 0)),
            pl.BlockSpec((TOK_BLOCK, H), lambda i, bps=blocks_per_seq: (i % bps, 0)),
            pl.BlockSpec((EXPR_PAD, H), lambda i: (0, 0)),
            pl.BlockSpec((1, H), lambda i: (0, 0)),
            pl.BlockSpec((1, H), lambda i: (0, 0)),
        ],
        out_specs=pl.BlockSpec((TOK_BLOCK, H), lambda i: (i, 0)),
        out_shape=jax.ShapeDtypeStruct((B * S, H), jnp.float32),
        compiler_params=pltpu.CompilerParams(
            dimension_semantics=("parallel",),
        ),
    )(ids3, g_tab, e_tab, ln_gamma.reshape(1, H), ln_beta.reshape(1, H))

    return out_flat.reshape(B, S, H)


# speedup vs baseline: 7.3252x; 1.1129x over previous
import functools

import jax
import jax.numpy as jnp
from jax.experimental import pallas as pl
from jax.experimental.pallas import tpu as pltpu

N_GENES = 100000
GENE_DIM = 64
EXPR_PAD = 64
HIDDEN = 128
BATCH = 64
SEQ = 2048
TOK_BLOCK = 4096


def _precompute_kernel(gene_ref, expr_ref, w_ref, b_ref, pos_ref, g_out_ref, e_out_ref):
    w1 = w_ref[0:GENE_DIM, :]
    w2 = w_ref[GENE_DIM : 2 * GENE_DIM, :]
    g = jax.lax.dot_general(
        gene_ref[...], w1, (((1,), (0,)), ((), ())),
        preferred_element_type=jnp.float32,
    )
    g_out_ref[...] = g + b_ref[...] + pos_ref[...]
    e_out_ref[...] = jax.lax.dot_general(
        expr_ref[...], w2, (((1,), (0,)), ((), ())),
        preferred_element_type=jnp.float32,
    )


def _main_kernel(ids_ref, g_ref, e_ref, gamma_ref, beta_ref, out_ref):
    ids = ids_ref[0, :, :]
    ids_b = jnp.broadcast_to(ids, (EXPR_PAD, TOK_BLOCK))
    vocab_iota = jax.lax.broadcasted_iota(jnp.int32, (EXPR_PAD, TOK_BLOCK), 0)
    onehot = (ids_b == vocab_iota).astype(jnp.float32)
    gathered = jax.lax.dot_general(
        onehot, e_ref[...], (((0,), (0,)), ((), ())),
        preferred_element_type=jnp.float32,
    )
    k = TOK_BLOCK // SEQ
    x = gathered.reshape(k, SEQ, HIDDEN) + g_ref[...][None, :, :]
    x = x.reshape(TOK_BLOCK, HIDDEN)
    mean = jnp.mean(x, axis=-1, keepdims=True)
    centered = x - mean
    var = jnp.mean(centered * centered, axis=-1, keepdims=True)
    y = centered * jax.lax.rsqrt(var + 1e-5)
    out_ref[...] = y * gamma_ref[...] + beta_ref[...]


@jax.jit
def kernel(input_ids, gene_table, expr_table, W_proj, b_proj, pos_table, ln_gamma, ln_beta):
    B, S = input_ids.shape
    V, D = expr_table.shape
    H = W_proj.shape[1]

    expr_pad = jnp.zeros((EXPR_PAD, D), dtype=expr_table.dtype).at[:V].set(expr_table)

    g_tab, e_tab = pl.pallas_call(
        _precompute_kernel,
        grid=(1,),
        in_specs=[
            pl.BlockSpec((S, D), lambda i: (0, 0)),
            pl.BlockSpec((EXPR_PAD, D), lambda i: (0, 0)),
            pl.BlockSpec((2 * D, H), lambda i: (0, 0)),
            pl.BlockSpec((1, H), lambda i: (0, 0)),
            pl.BlockSpec((S, H), lambda i: (0, 0)),
        ],
        out_specs=[
            pl.BlockSpec((S, H), lambda i: (0, 0)),
            pl.BlockSpec((EXPR_PAD, H), lambda i: (0, 0)),
        ],
        out_shape=[
            jax.ShapeDtypeStruct((S, H), jnp.float32),
            jax.ShapeDtypeStruct((EXPR_PAD, H), jnp.float32),
        ],
    )(gene_table, expr_pad, W_proj, b_proj.reshape(1, H), pos_table)

    n_blocks = (B * S) // TOK_BLOCK
    ids3 = input_ids.astype(jnp.int32).reshape(n_blocks, 1, TOK_BLOCK)

    out_flat = pl.pallas_call(
        _main_kernel,
        grid=(n_blocks,),
        in_specs=[
            pl.BlockSpec((1, 1, TOK_BLOCK), lambda i: (i, 0, 0)),
            pl.BlockSpec((S, H), lambda i: (0, 0)),
            pl.BlockSpec((EXPR_PAD, H), lambda i: (0, 0)),
            pl.BlockSpec((1, H), lambda i: (0, 0)),
            pl.BlockSpec((1, H), lambda i: (0, 0)),
        ],
        out_specs=pl.BlockSpec((TOK_BLOCK, H), lambda i: (i, 0)),
        out_shape=jax.ShapeDtypeStruct((B * S, H), jnp.float32),
        compiler_params=pltpu.CompilerParams(
            dimension_semantics=("parallel",),
        ),
    )(ids3, g_tab, e_tab, ln_gamma.reshape(1, H), ln_beta.reshape(1, H))

    return out_flat.reshape(B, S, H)


# speedup vs baseline: 7.4385x; 1.0155x over previous
import functools

import jax
import jax.numpy as jnp
from jax.experimental import pallas as pl
from jax.experimental.pallas import tpu as pltpu

N_GENES = 100000
GENE_DIM = 64
EXPR_PAD = 64
HIDDEN = 128
BATCH = 64
SEQ = 2048
TOK_BLOCK = 8192


def _precompute_kernel(gene_ref, expr_ref, w_ref, b_ref, pos_ref, g_out_ref, e_out_ref):
    w1 = w_ref[0:GENE_DIM, :]
    w2 = w_ref[GENE_DIM : 2 * GENE_DIM, :]
    g = jax.lax.dot_general(
        gene_ref[...], w1, (((1,), (0,)), ((), ())),
        preferred_element_type=jnp.float32,
    )
    g_out_ref[...] = g + b_ref[...] + pos_ref[...]
    e_out_ref[...] = jax.lax.dot_general(
        expr_ref[...], w2, (((1,), (0,)), ((), ())),
        preferred_element_type=jnp.float32,
    )


def _main_kernel(ids_ref, g_ref, e_ref, gamma_ref, beta_ref, out_ref):
    ids = ids_ref[0, :, :]
    ids_b = jnp.broadcast_to(ids, (EXPR_PAD, TOK_BLOCK))
    vocab_iota = jax.lax.broadcasted_iota(jnp.int32, (EXPR_PAD, TOK_BLOCK), 0)
    onehot = (ids_b == vocab_iota).astype(jnp.float32)
    gathered = jax.lax.dot_general(
        onehot, e_ref[...], (((0,), (0,)), ((), ())),
        preferred_element_type=jnp.float32,
    )
    k = TOK_BLOCK // SEQ
    x = gathered.reshape(k, SEQ, HIDDEN) + g_ref[...][None, :, :]
    x = x.reshape(TOK_BLOCK, HIDDEN)
    mean = jnp.mean(x, axis=-1, keepdims=True)
    centered = x - mean
    var = jnp.mean(centered * centered, axis=-1, keepdims=True)
    y = centered * jax.lax.rsqrt(var + 1e-5)
    out_ref[...] = y * gamma_ref[...] + beta_ref[...]


@jax.jit
def kernel(input_ids, gene_table, expr_table, W_proj, b_proj, pos_table, ln_gamma, ln_beta):
    B, S = input_ids.shape
    V, D = expr_table.shape
    H = W_proj.shape[1]

    expr_pad = jnp.zeros((EXPR_PAD, D), dtype=expr_table.dtype).at[:V].set(expr_table)

    g_tab, e_tab = pl.pallas_call(
        _precompute_kernel,
        grid=(1,),
        in_specs=[
            pl.BlockSpec((S, D), lambda i: (0, 0)),
            pl.BlockSpec((EXPR_PAD, D), lambda i: (0, 0)),
            pl.BlockSpec((2 * D, H), lambda i: (0, 0)),
            pl.BlockSpec((1, H), lambda i: (0, 0)),
            pl.BlockSpec((S, H), lambda i: (0, 0)),
        ],
        out_specs=[
            pl.BlockSpec((S, H), lambda i: (0, 0)),
            pl.BlockSpec((EXPR_PAD, H), lambda i: (0, 0)),
        ],
        out_shape=[
            jax.ShapeDtypeStruct((S, H), jnp.float32),
            jax.ShapeDtypeStruct((EXPR_PAD, H), jnp.float32),
        ],
    )(gene_table, expr_pad, W_proj, b_proj.reshape(1, H), pos_table)

    n_blocks = (B * S) // TOK_BLOCK
    ids3 = input_ids.astype(jnp.int32).reshape(n_blocks, 1, TOK_BLOCK)

    out_flat = pl.pallas_call(
        _main_kernel,
        grid=(n_blocks,),
        in_specs=[
            pl.BlockSpec((1, 1, TOK_BLOCK), lambda i: (i, 0, 0)),
            pl.BlockSpec((S, H), lambda i: (0, 0)),
            pl.BlockSpec((EXPR_PAD, H), lambda i: (0, 0)),
            pl.BlockSpec((1, H), lambda i: (0, 0)),
            pl.BlockSpec((1, H), lambda i: (0, 0)),
        ],
        out_specs=pl.BlockSpec((TOK_BLOCK, H), lambda i: (i, 0)),
        out_shape=jax.ShapeDtypeStruct((B * S, H), jnp.float32),
        compiler_params=pltpu.CompilerParams(
            dimension_semantics=("parallel",),
        ),
    )(ids3, g_tab, e_tab, ln_gamma.reshape(1, H), ln_beta.reshape(1, H))

    return out_flat.reshape(B, S, H)
